# Initial kernel scaffold; baseline (speedup 1.0000x reference)
import jax
import jax.numpy as jnp
from jax.experimental import pallas as pl


def kernel(h, e, edge_index, W_q, W_k, W_e, attn_vec, W_self, b_self, W_msg, W_eu, b_eu, ln_gamma, ln_beta):
    raise NotImplementedError("write your pallas kernel here")


# speedup vs baseline: 3.7899x; 3.7899x over previous
import functools

import jax
import jax.numpy as jnp
from jax import lax
from jax.experimental import pallas as pl
from jax.experimental.pallas import tpu as pltpu, tpu_sc as plsc

N_NODES = 10000
N_EDGES = 320000
D = 128
ED = 16

_NC = 2
_NS = 16
_NW = _NC * _NS
_CH = 128
_NCHUNKS = N_EDGES // _CH
_BASE_CHUNKS = _NCHUNKS // _NW
_EXTRA = _NCHUNKS - _BASE_CHUNKS * _NW

_ROWS_PER_SUB = 624
_SUM_PER_SUB = 1000

_PREC = lax.Precision.HIGHEST


def _lrelu(x):
    return jnp.where(x >= 0, x, 0.2 * x)


def _dot_t(a, b):
    return lax.dot_general(a, b, (((1,), (1,)), ((), ())),
                           precision=_PREC, preferred_element_type=jnp.float32)


def _tc_node_body(h, wq, wk, wmsg, wself, bself, avq, avk,
                  aq_o, ak_o, m_o, s_o):
    hh = h[...]
    q = _dot_t(hh, wq[...])
    aq_o[...] = jnp.dot(_lrelu(q), avq[...], precision=_PREC,
                        preferred_element_type=jnp.float32)
    k = _dot_t(hh, wk[...])
    ak_o[...] = jnp.dot(_lrelu(k), avk[...], precision=_PREC,
                        preferred_element_type=jnp.float32)
    m_o[...] = _dot_t(hh, wmsg[...])
    s_o[...] = _dot_t(hh, wself[...]) + bself[...]


def _tc_node(h, wq, wk, wmsg, wself, bself, avq, avk):
    blk = 2000
    grid = N_NODES // blk
    return pl.pallas_call(
        _tc_node_body,
        grid=(grid,),
        in_specs=[
            pl.BlockSpec((blk, D), lambda i: (i, 0)),
            pl.BlockSpec((D, D), lambda i: (0, 0)),
            pl.BlockSpec((D, D), lambda i: (0, 0)),
            pl.BlockSpec((D, D), lambda i: (0, 0)),
            pl.BlockSpec((D, D), lambda i: (0, 0)),
            pl.BlockSpec((1, D), lambda i: (0, 0)),
            pl.BlockSpec((D, 1), lambda i: (0, 0)),
            pl.BlockSpec((D, 1), lambda i: (0, 0)),
        ],
        out_specs=[
            pl.BlockSpec((blk, 1), lambda i: (i, 0)),
            pl.BlockSpec((blk, 1), lambda i: (i, 0)),
            pl.BlockSpec((blk, D), lambda i: (i, 0)),
            pl.BlockSpec((blk, D), lambda i: (i, 0)),
        ],
        out_shape=[
            jax.ShapeDtypeStruct((N_NODES, 1), jnp.float32),
            jax.ShapeDtypeStruct((N_NODES, 1), jnp.float32),
            jax.ShapeDtypeStruct((N_NODES, D), jnp.float32),
            jax.ShapeDtypeStruct((N_NODES, D), jnp.float32),
        ],
    )(h, wq, wk, wmsg, wself, bself, avq, avk)


def _tc_edge_body(e, we, ave, w3, beu, ae_o, e3_o):
    eb = e[...]
    p = _dot_t(eb, we[...])
    ae_o[...] = jnp.dot(_lrelu(p), ave[...], precision=_PREC,
                        preferred_element_type=jnp.float32)
    e3_o[...] = _dot_t(eb, w3[...]) + beu[...]


def _tc_edge(e, we, ave, w3, beu):
    blk = 8000
    grid = N_EDGES // blk
    return pl.pallas_call(
        _tc_edge_body,
        grid=(grid,),
        in_specs=[
            pl.BlockSpec((blk, ED), lambda i: (i, 0)),
            pl.BlockSpec((D, ED), lambda i: (0, 0)),
            pl.BlockSpec((D, 1), lambda i: (0, 0)),
            pl.BlockSpec((ED, ED), lambda i: (0, 0)),
            pl.BlockSpec((1, ED), lambda i: (0, 0)),
        ],
        out_specs=[
            pl.BlockSpec((blk, 1), lambda i: (i, 0)),
            pl.BlockSpec((blk, ED), lambda i: (i, 0)),
        ],
        out_shape=[
            jax.ShapeDtypeStruct((N_EDGES, 1), jnp.float32),
            jax.ShapeDtypeStruct((N_EDGES, ED), jnp.float32),
        ],
    )(e, we, ave, w3, beu)


def _sc_agg_body(aq_h, ak_h, ae_h, m_h, src_h, dst_h,
                 aggp_o, sump_o,
                 srcv, dstv, aev, aqv, akv, pv, rowsv, zbuf, agg_sh, sum_sh,
                 sem):
    c = lax.axis_index("c")
    s = lax.axis_index("s")
    wid = s * _NC + c

    def _zrow(r, carry):
        for jj in range(8):
            rowsv[r, pl.ds(jj * 16, 16)] = jnp.zeros((16,), jnp.float32)
        return carry
    lax.fori_loop(0, _CH, _zrow, 0)

    def _zb(i, carry):
        zbuf[pl.ds(i * 16, 16)] = jnp.zeros((16,), jnp.float32)
        return carry
    lax.fori_loop(0, 64, _zb, 0)

    for k in range(5):
        nr = _CH if k < 4 else (_ROWS_PER_SUB - 4 * _CH)
        pltpu.sync_copy(rowsv.at[pl.ds(0, nr)],
                        agg_sh.at[pl.ds(s * _ROWS_PER_SUB + k * _CH, nr)])

    @pl.when(s == _NS - 1)
    def _():
        pltpu.sync_copy(rowsv.at[pl.ds(0, N_NODES - _NS * _ROWS_PER_SUB)],
                        agg_sh.at[pl.ds(_NS * _ROWS_PER_SUB,
                                        N_NODES - _NS * _ROWS_PER_SUB)])

    @pl.when(s < N_NODES // _SUM_PER_SUB)
    def _():
        pltpu.sync_copy(zbuf.at[pl.ds(0, 1000)],
                        sum_sh.at[pl.ds(s * _SUM_PER_SUB, _SUM_PER_SUB)])

    plsc.subcore_barrier()

    nj = _BASE_CHUNKS + jnp.where(wid < _EXTRA, 1, 0)

    def _chunk(j, carry):
        base = (wid + _NW * j) * _CH
        pltpu.sync_copy(src_h.at[pl.ds(base, _CH)], srcv)
        pltpu.sync_copy(dst_h.at[pl.ds(base, _CH)], dstv)
        pltpu.sync_copy(ae_h.at[pl.ds(base, _CH)], aev)
        pltpu.async_copy(aq_h.at[dstv], aqv, sem).wait()
        pltpu.async_copy(ak_h.at[srcv], akv, sem).wait()
        pltpu.async_copy(m_h.at[srcv], rowsv, sem).wait()
        for jj in range(8):
            sl = pl.ds(jj * 16, 16)
            pv[sl] = jnp.exp(aqv[sl] + akv[sl] + aev[sl])

        def _scale(g, carry2):
            pvec = pv[pl.ds(g * 16, 16)]
            for u in range(16):
                pr = pvec[u]
                r = g * 16 + u
                for jj in range(8):
                    sl = pl.ds(jj * 16, 16)
                    rowsv[r, sl] = rowsv[r, sl] * pr
            return carry2
        lax.fori_loop(0, _CH // 16, _scale, 0)

        pltpu.sync_copy(rowsv, agg_sh.at[dstv], add=True)
        pltpu.sync_copy(pv, sum_sh.at[dstv], add=True)
        return carry
    lax.fori_loop(0, nj, _chunk, 0)

    plsc.subcore_barrier()

    for k in range(5):
        nr = _CH if k < 4 else (_ROWS_PER_SUB - 4 * _CH)
        r0 = s * _ROWS_PER_SUB + k * _CH
        pltpu.sync_copy(agg_sh.at[pl.ds(r0, nr)], rowsv.at[pl.ds(0, nr)])
        pltpu.sync_copy(rowsv.at[pl.ds(0, nr)], aggp_o.at[c, pl.ds(r0, nr)])

    _REM = N_NODES - _NS * _ROWS_PER_SUB

    @pl.when(s == _NS - 1)
    def _():
        pltpu.sync_copy(agg_sh.at[pl.ds(_NS * _ROWS_PER_SUB, _REM)],
                        rowsv.at[pl.ds(0, _REM)])
        pltpu.sync_copy(rowsv.at[pl.ds(0, _REM)],
                        aggp_o.at[c, pl.ds(_NS * _ROWS_PER_SUB, _REM)])

    @pl.when(s < N_NODES // _SUM_PER_SUB)
    def _():
        pltpu.sync_copy(sum_sh.at[pl.ds(s * _SUM_PER_SUB, _SUM_PER_SUB)],
                        zbuf.at[pl.ds(0, _SUM_PER_SUB)])
        pltpu.sync_copy(
            zbuf.at[pl.ds(0, _SUM_PER_SUB)],
            sump_o.at[pl.ds(c * N_NODES + s * _SUM_PER_SUB, _SUM_PER_SUB)])


def _sc_agg(aq, ak, ae, m, src, dst):
    mesh = plsc.VectorSubcoreMesh(core_axis_name="c", subcore_axis_name="s")
    f = functools.partial(
        pl.kernel,
        mesh=mesh,
        out_type=[
            jax.ShapeDtypeStruct((_NC, N_NODES, D), jnp.float32),
            jax.ShapeDtypeStruct((_NC * N_NODES,), jnp.float32),
        ],
        scratch_types=[
            pltpu.VMEM((_CH,), jnp.int32),
            pltpu.VMEM((_CH,), jnp.int32),
            pltpu.VMEM((_CH,), jnp.float32),
            pltpu.VMEM((_CH,), jnp.float32),
            pltpu.VMEM((_CH,), jnp.float32),
            pltpu.VMEM((_CH,), jnp.float32),
            pltpu.VMEM((_CH, D), jnp.float32),
            pltpu.VMEM((1024,), jnp.float32),
            pltpu.VMEM_SHARED((N_NODES, D), jnp.float32),
            pltpu.VMEM_SHARED((N_NODES,), jnp.float32),
            pltpu.SemaphoreType.DMA,
        ],
    )(_sc_agg_body)
    return f(aq, ak, ae, m, src, dst)


def _tc_final_body(aggp, sump, s_in, h, gamma, beta, w1, w2,
                   hn_o, u1_o, u2_o):
    denom = sump[0] + sump[1] + 1e-12
    agg = (aggp[0] + aggp[1]) / denom
    pre = jnp.maximum(s_in[...] + agg, 0.0) + h[...]
    mean = jnp.mean(pre, axis=-1, keepdims=True)
    cen = pre - mean
    var = jnp.mean(cen * cen, axis=-1, keepdims=True)
    hn = cen / jnp.sqrt(var + 1e-5) * gamma[...] + beta[...]
    hn_o[...] = hn
    u1_o[...] = _dot_t(hn, w1[...])
    u2_o[...] = _dot_t(hn, w2[...])


def _tc_final(aggp, sump, s_in, h, gamma, beta, w1, w2):
    blk = 2000
    grid = N_NODES // blk
    return pl.pallas_call(
        _tc_final_body,
        grid=(grid,),
        in_specs=[
            pl.BlockSpec((_NC, blk, D), lambda i: (0, i, 0)),
            pl.BlockSpec((_NC, blk, 1), lambda i: (0, i, 0)),
            pl.BlockSpec((blk, D), lambda i: (i, 0)),
            pl.BlockSpec((blk, D), lambda i: (i, 0)),
            pl.BlockSpec((1, D), lambda i: (0, 0)),
            pl.BlockSpec((1, D), lambda i: (0, 0)),
            pl.BlockSpec((D, D), lambda i: (0, 0)),
            pl.BlockSpec((D, D), lambda i: (0, 0)),
        ],
        out_specs=[
            pl.BlockSpec((blk, D), lambda i: (i, 0)),
            pl.BlockSpec((blk, D), lambda i: (i, 0)),
            pl.BlockSpec((blk, D), lambda i: (i, 0)),
        ],
        out_shape=[
            jax.ShapeDtypeStruct((N_NODES, D), jnp.float32),
            jax.ShapeDtypeStruct((N_NODES, D), jnp.float32),
            jax.ShapeDtypeStruct((N_NODES, D), jnp.float32),
        ],
    )(aggp, sump, s_in, h, gamma, beta, w1, w2)


def _sc_edge_body(u1_h, u2_h, e3_h, e_h, src_h, dst_h, enew_o,
                  srcv, dstv, u1v, u2v, e3v, ev, outv, sem):
    c = lax.axis_index("c")
    s = lax.axis_index("s")
    wid = s * _NC + c
    nj = _BASE_CHUNKS + jnp.where(wid < _EXTRA, 1, 0)

    def _chunk(j, carry):
        base = (wid + _NW * j) * _CH
        pltpu.sync_copy(src_h.at[pl.ds(base, _CH)], srcv)
        pltpu.sync_copy(dst_h.at[pl.ds(base, _CH)], dstv)
        pltpu.sync_copy(e3_h.at[pl.ds(base, _CH)], e3v)
        pltpu.sync_copy(e_h.at[pl.ds(base, _CH)], ev)
        pltpu.async_copy(u1_h.at[srcv], u1v, sem).wait()
        pltpu.async_copy(u2_h.at[dstv], u2v, sem).wait()

        def _row(r, carry2):
            sl = pl.ds(0, ED)
            outv[r] = jnp.maximum(u1v[r, sl] + u2v[r, sl] + e3v[r], 0.0) + ev[r]
            return carry2
        lax.fori_loop(0, _CH, _row, 0)

        pltpu.sync_copy(outv, enew_o.at[pl.ds(base, _CH)])
        return carry
    lax.fori_loop(0, nj, _chunk, 0)


def _sc_edge(u1, u2, e3, e, src, dst):
    mesh = plsc.VectorSubcoreMesh(core_axis_name="c", subcore_axis_name="s")
    f = functools.partial(
        pl.kernel,
        mesh=mesh,
        out_type=jax.ShapeDtypeStruct((N_EDGES, ED), jnp.float32),
        scratch_types=[
            pltpu.VMEM((_CH,), jnp.int32),
            pltpu.VMEM((_CH,), jnp.int32),
            pltpu.VMEM((_CH, D), jnp.float32),
            pltpu.VMEM((_CH, D), jnp.float32),
            pltpu.VMEM((_CH, ED), jnp.float32),
            pltpu.VMEM((_CH, ED), jnp.float32),
            pltpu.VMEM((_CH, ED), jnp.float32),
            pltpu.SemaphoreType.DMA,
        ],
    )(_sc_edge_body)
    return f(u1, u2, e3, e, src, dst)


def kernel(h, e, edge_index, W_q, W_k, W_e, attn_vec, W_self, b_self,
           W_msg, W_eu, b_eu, ln_gamma, ln_beta):
    src = edge_index[0]
    dst = edge_index[1]
    av = attn_vec[0]
    avq = av[:D].reshape(D, 1)
    avk = av[D:2 * D].reshape(D, 1)
    ave = av[2 * D:].reshape(D, 1)

    aq2, ak2, m, s_pre = _tc_node(h, W_q, W_k, W_msg, W_self,
                                  b_self.reshape(1, D), avq, avk)
    ae2, e3 = _tc_edge(e, W_e, ave, W_eu[:, 2 * D:], b_eu.reshape(1, ED))

    aggp, sump = _sc_agg(aq2.reshape(N_NODES), ak2.reshape(N_NODES),
                         ae2.reshape(N_EDGES), m, src, dst)

    w1p = jnp.zeros((D, D), jnp.float32).at[:ED].set(W_eu[:, :D])
    w2p = jnp.zeros((D, D), jnp.float32).at[:ED].set(W_eu[:, D:2 * D])
    h_new, u1, u2 = _tc_final(aggp, sump.reshape(_NC, N_NODES, 1), s_pre, h,
                              ln_gamma.reshape(1, D), ln_beta.reshape(1, D),
                              w1p, w2p)

    e_new = _sc_edge(u1, u2, e3, e, src, dst)
    return (h_new, e_new)


# speedup vs baseline: 4.9892x; 1.3164x over previous
import functools

import jax
import jax.numpy as jnp
from jax import lax
from jax.experimental import pallas as pl
from jax.experimental.pallas import tpu as pltpu, tpu_sc as plsc

N_NODES = 10000
N_EDGES = 320000
D = 128
ED = 16

_NC = 2
_NS = 16
_NW = _NC * _NS
_CH = 128
_NCHUNKS = N_EDGES // _CH
_BASE_CHUNKS = _NCHUNKS // _NW
_EXTRA = _NCHUNKS - _BASE_CHUNKS * _NW

_ROWS_PER_SUB = 624
_SUM_PER_SUB = 1000

_PREC = lax.Precision.DEFAULT


def _lrelu(x):
    return jnp.where(x >= 0, x, 0.2 * x)


def _dot_t(a, b):
    return lax.dot_general(a, b, (((1,), (1,)), ((), ())),
                           precision=_PREC, preferred_element_type=jnp.float32)


def _tc_node_body(h, wq, wk, wmsg, wself, bself, avq, avk,
                  aq_o, ak_o, m_o, s_o):
    hh = h[...]
    q = _dot_t(hh, wq[...])
    aq_o[...] = jnp.dot(_lrelu(q), avq[...], precision=_PREC,
                        preferred_element_type=jnp.float32)
    k = _dot_t(hh, wk[...])
    ak_o[...] = jnp.dot(_lrelu(k), avk[...], precision=_PREC,
                        preferred_element_type=jnp.float32)
    m_o[...] = _dot_t(hh, wmsg[...])
    s_o[...] = _dot_t(hh, wself[...]) + bself[...]


def _tc_node(h, wq, wk, wmsg, wself, bself, avq, avk):
    blk = 2000
    grid = N_NODES // blk
    return pl.pallas_call(
        _tc_node_body,
        grid=(grid,),
        in_specs=[
            pl.BlockSpec((blk, D), lambda i: (i, 0)),
            pl.BlockSpec((D, D), lambda i: (0, 0)),
            pl.BlockSpec((D, D), lambda i: (0, 0)),
            pl.BlockSpec((D, D), lambda i: (0, 0)),
            pl.BlockSpec((D, D), lambda i: (0, 0)),
            pl.BlockSpec((1, D), lambda i: (0, 0)),
            pl.BlockSpec((D, 1), lambda i: (0, 0)),
            pl.BlockSpec((D, 1), lambda i: (0, 0)),
        ],
        out_specs=[
            pl.BlockSpec((blk, 1), lambda i: (i, 0)),
            pl.BlockSpec((blk, 1), lambda i: (i, 0)),
            pl.BlockSpec((blk, D), lambda i: (i, 0)),
            pl.BlockSpec((blk, D), lambda i: (i, 0)),
        ],
        out_shape=[
            jax.ShapeDtypeStruct((N_NODES, 1), jnp.float32),
            jax.ShapeDtypeStruct((N_NODES, 1), jnp.float32),
            jax.ShapeDtypeStruct((N_NODES, D), jnp.float32),
            jax.ShapeDtypeStruct((N_NODES, D), jnp.float32),
        ],
    )(h, wq, wk, wmsg, wself, bself, avq, avk)


def _tc_edge_body(e, we, ave, w3, beu, ae_o, e3_o):
    eb = e[...]
    p = _dot_t(eb, we[...])
    ae_o[...] = jnp.dot(_lrelu(p), ave[...], precision=_PREC,
                        preferred_element_type=jnp.float32)
    e3_o[...] = _dot_t(eb, w3[...]) + beu[...]


def _tc_edge(e, we, ave, w3, beu):
    blk = 8000
    grid = N_EDGES // blk
    return pl.pallas_call(
        _tc_edge_body,
        grid=(grid,),
        in_specs=[
            pl.BlockSpec((blk, ED), lambda i: (i, 0)),
            pl.BlockSpec((D, ED), lambda i: (0, 0)),
            pl.BlockSpec((D, 1), lambda i: (0, 0)),
            pl.BlockSpec((ED, ED), lambda i: (0, 0)),
            pl.BlockSpec((1, ED), lambda i: (0, 0)),
        ],
        out_specs=[
            pl.BlockSpec((blk, 1), lambda i: (i, 0)),
            pl.BlockSpec((blk, ED), lambda i: (i, 0)),
        ],
        out_shape=[
            jax.ShapeDtypeStruct((N_EDGES, 1), jnp.float32),
            jax.ShapeDtypeStruct((N_EDGES, ED), jnp.float32),
        ],
    )(e, we, ave, w3, beu)


def _sc_agg_body(aq_h, ak_h, ae_h, m_h, src_h, dst_h,
                 aggp_o, sump_o,
                 srcv, dstv, aev, aqv, akv, pv, rowsv, zbuf, agg_sh, sum_sh,
                 sem):
    c = lax.axis_index("c")
    s = lax.axis_index("s")
    wid = s * _NC + c

    def _zrow(r, carry):
        for jj in range(8):
            rowsv[r, pl.ds(jj * 16, 16)] = jnp.zeros((16,), jnp.float32)
        return carry
    lax.fori_loop(0, _CH, _zrow, 0)

    def _zb(i, carry):
        zbuf[pl.ds(i * 16, 16)] = jnp.zeros((16,), jnp.float32)
        return carry
    lax.fori_loop(0, 64, _zb, 0)

    for k in range(5):
        nr = _CH if k < 4 else (_ROWS_PER_SUB - 4 * _CH)
        pltpu.sync_copy(rowsv.at[pl.ds(0, nr)],
                        agg_sh.at[pl.ds(s * _ROWS_PER_SUB + k * _CH, nr)])

    @pl.when(s == _NS - 1)
    def _():
        pltpu.sync_copy(rowsv.at[pl.ds(0, N_NODES - _NS * _ROWS_PER_SUB)],
                        agg_sh.at[pl.ds(_NS * _ROWS_PER_SUB,
                                        N_NODES - _NS * _ROWS_PER_SUB)])

    @pl.when(s < N_NODES // _SUM_PER_SUB)
    def _():
        pltpu.sync_copy(zbuf.at[pl.ds(0, 1000)],
                        sum_sh.at[pl.ds(s * _SUM_PER_SUB, _SUM_PER_SUB)])

    plsc.subcore_barrier()

    nj = _BASE_CHUNKS + jnp.where(wid < _EXTRA, 1, 0)

    def _chunk(j, carry):
        base = (wid + _NW * j) * _CH
        pltpu.sync_copy(src_h.at[pl.ds(base, _CH)], srcv)
        pltpu.sync_copy(dst_h.at[pl.ds(base, _CH)], dstv)
        pltpu.sync_copy(ae_h.at[pl.ds(base, _CH)], aev)
        pltpu.async_copy(aq_h.at[dstv], aqv, sem).wait()
        pltpu.async_copy(ak_h.at[srcv], akv, sem).wait()
        pltpu.async_copy(m_h.at[srcv], rowsv, sem).wait()
        for jj in range(8):
            sl = pl.ds(jj * 16, 16)
            pv[sl] = jnp.exp(aqv[sl] + akv[sl] + aev[sl])

        def _scale(g, carry2):
            pvec = pv[pl.ds(g * 16, 16)]
            for u in range(16):
                pr = pvec[u]
                r = g * 16 + u
                for jj in range(8):
                    sl = pl.ds(jj * 16, 16)
                    rowsv[r, sl] = rowsv[r, sl] * pr
            return carry2
        lax.fori_loop(0, _CH // 16, _scale, 0)

        pltpu.sync_copy(rowsv, agg_sh.at[dstv], add=True)
        pltpu.sync_copy(pv, sum_sh.at[dstv], add=True)
        return carry
    lax.fori_loop(0, nj, _chunk, 0)

    plsc.subcore_barrier()

    for k in range(5):
        nr = _CH if k < 4 else (_ROWS_PER_SUB - 4 * _CH)
        r0 = s * _ROWS_PER_SUB + k * _CH
        pltpu.sync_copy(agg_sh.at[pl.ds(r0, nr)], rowsv.at[pl.ds(0, nr)])
        pltpu.sync_copy(rowsv.at[pl.ds(0, nr)], aggp_o.at[c, pl.ds(r0, nr)])

    _REM = N_NODES - _NS * _ROWS_PER_SUB

    @pl.when(s == _NS - 1)
    def _():
        pltpu.sync_copy(agg_sh.at[pl.ds(_NS * _ROWS_PER_SUB, _REM)],
                        rowsv.at[pl.ds(0, _REM)])
        pltpu.sync_copy(rowsv.at[pl.ds(0, _REM)],
                        aggp_o.at[c, pl.ds(_NS * _ROWS_PER_SUB, _REM)])

    @pl.when(s < N_NODES // _SUM_PER_SUB)
    def _():
        pltpu.sync_copy(sum_sh.at[pl.ds(s * _SUM_PER_SUB, _SUM_PER_SUB)],
                        zbuf.at[pl.ds(0, _SUM_PER_SUB)])
        pltpu.sync_copy(
            zbuf.at[pl.ds(0, _SUM_PER_SUB)],
            sump_o.at[pl.ds(c * N_NODES + s * _SUM_PER_SUB, _SUM_PER_SUB)])


def _sc_agg(aq, ak, ae, m, src, dst):
    mesh = plsc.VectorSubcoreMesh(core_axis_name="c", subcore_axis_name="s")
    f = functools.partial(
        pl.kernel,
        mesh=mesh,
        out_type=[
            jax.ShapeDtypeStruct((_NC, N_NODES, D), jnp.float32),
            jax.ShapeDtypeStruct((_NC * N_NODES,), jnp.float32),
        ],
        scratch_types=[
            pltpu.VMEM((_CH,), jnp.int32),
            pltpu.VMEM((_CH,), jnp.int32),
            pltpu.VMEM((_CH,), jnp.float32),
            pltpu.VMEM((_CH,), jnp.float32),
            pltpu.VMEM((_CH,), jnp.float32),
            pltpu.VMEM((_CH,), jnp.float32),
            pltpu.VMEM((_CH, D), jnp.float32),
            pltpu.VMEM((1024,), jnp.float32),
            pltpu.VMEM_SHARED((N_NODES, D), jnp.float32),
            pltpu.VMEM_SHARED((N_NODES,), jnp.float32),
            pltpu.SemaphoreType.DMA,
        ],
    )(_sc_agg_body)
    return f(aq, ak, ae, m, src, dst)


def _tc_final_body(aggp, sump, s_in, h, gamma, beta, w1, w2,
                   hn_o, u1_o, u2_o):
    denom = sump[0] + sump[1] + 1e-12
    agg = (aggp[0] + aggp[1]) / denom
    pre = jnp.maximum(s_in[...] + agg, 0.0) + h[...]
    mean = jnp.mean(pre, axis=-1, keepdims=True)
    cen = pre - mean
    var = jnp.mean(cen * cen, axis=-1, keepdims=True)
    hn = cen / jnp.sqrt(var + 1e-5) * gamma[...] + beta[...]
    hn_o[...] = hn
    u1_o[...] = _dot_t(hn, w1[...])
    u2_o[...] = _dot_t(hn, w2[...])


def _tc_final(aggp, sump, s_in, h, gamma, beta, w1, w2):
    blk = 2000
    grid = N_NODES // blk
    return pl.pallas_call(
        _tc_final_body,
        grid=(grid,),
        in_specs=[
            pl.BlockSpec((_NC, blk, D), lambda i: (0, i, 0)),
            pl.BlockSpec((_NC, blk, 1), lambda i: (0, i, 0)),
            pl.BlockSpec((blk, D), lambda i: (i, 0)),
            pl.BlockSpec((blk, D), lambda i: (i, 0)),
            pl.BlockSpec((1, D), lambda i: (0, 0)),
            pl.BlockSpec((1, D), lambda i: (0, 0)),
            pl.BlockSpec((D, D), lambda i: (0, 0)),
            pl.BlockSpec((D, D), lambda i: (0, 0)),
        ],
        out_specs=[
            pl.BlockSpec((blk, D), lambda i: (i, 0)),
            pl.BlockSpec((blk, D), lambda i: (i, 0)),
            pl.BlockSpec((blk, D), lambda i: (i, 0)),
        ],
        out_shape=[
            jax.ShapeDtypeStruct((N_NODES, D), jnp.float32),
            jax.ShapeDtypeStruct((N_NODES, D), jnp.float32),
            jax.ShapeDtypeStruct((N_NODES, D), jnp.float32),
        ],
    )(aggp, sump, s_in, h, gamma, beta, w1, w2)


def _sc_edge_body(u1_h, u2_h, e3_h, e_h, src_h, dst_h, enew_o,
                  srcv, dstv, u1v, u2v, e3v, ev, outv, sem):
    c = lax.axis_index("c")
    s = lax.axis_index("s")
    wid = s * _NC + c
    nj = _BASE_CHUNKS + jnp.where(wid < _EXTRA, 1, 0)

    def _chunk(j, carry):
        base = (wid + _NW * j) * _CH
        pltpu.sync_copy(src_h.at[pl.ds(base, _CH)], srcv)
        pltpu.sync_copy(dst_h.at[pl.ds(base, _CH)], dstv)
        pltpu.sync_copy(e3_h.at[pl.ds(base, _CH)], e3v)
        pltpu.sync_copy(e_h.at[pl.ds(base, _CH)], ev)
        pltpu.async_copy(u1_h.at[srcv], u1v, sem).wait()
        pltpu.async_copy(u2_h.at[dstv], u2v, sem).wait()

        def _row(r, carry2):
            sl = pl.ds(0, ED)
            outv[r] = jnp.maximum(u1v[r, sl] + u2v[r, sl] + e3v[r], 0.0) + ev[r]
            return carry2
        lax.fori_loop(0, _CH, _row, 0)

        pltpu.sync_copy(outv, enew_o.at[pl.ds(base, _CH)])
        return carry
    lax.fori_loop(0, nj, _chunk, 0)


def _sc_edge(u1, u2, e3, e, src, dst):
    mesh = plsc.VectorSubcoreMesh(core_axis_name="c", subcore_axis_name="s")
    f = functools.partial(
        pl.kernel,
        mesh=mesh,
        out_type=jax.ShapeDtypeStruct((N_EDGES, ED), jnp.float32),
        scratch_types=[
            pltpu.VMEM((_CH,), jnp.int32),
            pltpu.VMEM((_CH,), jnp.int32),
            pltpu.VMEM((_CH, D), jnp.float32),
            pltpu.VMEM((_CH, D), jnp.float32),
            pltpu.VMEM((_CH, ED), jnp.float32),
            pltpu.VMEM((_CH, ED), jnp.float32),
            pltpu.VMEM((_CH, ED), jnp.float32),
            pltpu.SemaphoreType.DMA,
        ],
    )(_sc_edge_body)
    return f(u1, u2, e3, e, src, dst)


def kernel(h, e, edge_index, W_q, W_k, W_e, attn_vec, W_self, b_self,
           W_msg, W_eu, b_eu, ln_gamma, ln_beta):
    src = edge_index[0]
    dst = edge_index[1]
    av = attn_vec[0]
    avq = av[:D].reshape(D, 1)
    avk = av[D:2 * D].reshape(D, 1)
    ave = av[2 * D:].reshape(D, 1)

    aq2, ak2, m, s_pre = _tc_node(h, W_q, W_k, W_msg, W_self,
                                  b_self.reshape(1, D), avq, avk)
    ae2, e3 = _tc_edge(e, W_e, ave, W_eu[:, 2 * D:], b_eu.reshape(1, ED))

    aggp, sump = _sc_agg(aq2.reshape(N_NODES), ak2.reshape(N_NODES),
                         ae2.reshape(N_EDGES), m, src, dst)

    w1p = jnp.zeros((D, D), jnp.float32).at[:ED].set(W_eu[:, :D])
    w2p = jnp.zeros((D, D), jnp.float32).at[:ED].set(W_eu[:, D:2 * D])
    h_new, u1, u2 = _tc_final(aggp, sump.reshape(_NC, N_NODES, 1), s_pre, h,
                              ln_gamma.reshape(1, D), ln_beta.reshape(1, D),
                              w1p, w2p)

    e_new = _sc_edge(u1, u2, e3, e, src, dst)
    return (h_new, e_new)


# speedup vs baseline: 5.5982x; 1.1221x over previous
import functools

import jax
import jax.numpy as jnp
from jax import lax
from jax.experimental import pallas as pl
from jax.experimental.pallas import tpu as pltpu, tpu_sc as plsc

N_NODES = 10000
N_EDGES = 320000
D = 128
ED = 16

_NC = 2
_NS = 16
_NW = _NC * _NS
_CH = 128
_NCHUNKS = N_EDGES // _CH
_BASE_CHUNKS = _NCHUNKS // _NW
_EXTRA = _NCHUNKS - _BASE_CHUNKS * _NW

_ROWS_PER_SUB = 624
_SUM_PER_SUB = 1000

_PREC = lax.Precision.DEFAULT


def _lrelu(x):
    return jnp.where(x >= 0, x, 0.2 * x)


def _dot_t(a, b):
    return lax.dot_general(a, b, (((1,), (1,)), ((), ())),
                           precision=_PREC, preferred_element_type=jnp.float32)


def _tc_node_body(h, wq, wk, wmsg, wself, bself, avq, avk,
                  aq_o, ak_o, m_o, s_o):
    hh = h[...]
    q = _dot_t(hh, wq[...])
    aq_o[...] = jnp.dot(_lrelu(q), avq[...], precision=_PREC,
                        preferred_element_type=jnp.float32)
    k = _dot_t(hh, wk[...])
    ak_o[...] = jnp.dot(_lrelu(k), avk[...], precision=_PREC,
                        preferred_element_type=jnp.float32)
    m_o[...] = _dot_t(hh, wmsg[...])
    s_o[...] = _dot_t(hh, wself[...]) + bself[...]


def _tc_node(h, wq, wk, wmsg, wself, bself, avq, avk):
    blk = 2000
    grid = N_NODES // blk
    return pl.pallas_call(
        _tc_node_body,
        grid=(grid,),
        in_specs=[
            pl.BlockSpec((blk, D), lambda i: (i, 0)),
            pl.BlockSpec((D, D), lambda i: (0, 0)),
            pl.BlockSpec((D, D), lambda i: (0, 0)),
            pl.BlockSpec((D, D), lambda i: (0, 0)),
            pl.BlockSpec((D, D), lambda i: (0, 0)),
            pl.BlockSpec((1, D), lambda i: (0, 0)),
            pl.BlockSpec((D, 1), lambda i: (0, 0)),
            pl.BlockSpec((D, 1), lambda i: (0, 0)),
        ],
        out_specs=[
            pl.BlockSpec((blk, 1), lambda i: (i, 0)),
            pl.BlockSpec((blk, 1), lambda i: (i, 0)),
            pl.BlockSpec((blk, D), lambda i: (i, 0)),
            pl.BlockSpec((blk, D), lambda i: (i, 0)),
        ],
        out_shape=[
            jax.ShapeDtypeStruct((N_NODES, 1), jnp.float32),
            jax.ShapeDtypeStruct((N_NODES, 1), jnp.float32),
            jax.ShapeDtypeStruct((N_NODES, D), jnp.float32),
            jax.ShapeDtypeStruct((N_NODES, D), jnp.float32),
        ],
    )(h, wq, wk, wmsg, wself, bself, avq, avk)


def _tc_edge_body(e, we, ave, w3, beu, ae_o, e3_o):
    eb = e[...]
    p = _dot_t(eb, we[...])
    ae_o[...] = jnp.dot(_lrelu(p), ave[...], precision=_PREC,
                        preferred_element_type=jnp.float32)
    e3_o[...] = _dot_t(eb, w3[...]) + beu[...]


def _tc_edge(e, we, ave, w3, beu):
    blk = 8000
    grid = N_EDGES // blk
    return pl.pallas_call(
        _tc_edge_body,
        grid=(grid,),
        in_specs=[
            pl.BlockSpec((blk, ED), lambda i: (i, 0)),
            pl.BlockSpec((D, ED), lambda i: (0, 0)),
            pl.BlockSpec((D, 1), lambda i: (0, 0)),
            pl.BlockSpec((ED, ED), lambda i: (0, 0)),
            pl.BlockSpec((1, ED), lambda i: (0, 0)),
        ],
        out_specs=[
            pl.BlockSpec((blk, 1), lambda i: (i, 0)),
            pl.BlockSpec((blk, ED), lambda i: (i, 0)),
        ],
        out_shape=[
            jax.ShapeDtypeStruct((N_EDGES, 1), jnp.float32),
            jax.ShapeDtypeStruct((N_EDGES, ED), jnp.float32),
        ],
    )(e, we, ave, w3, beu)


def _sc_agg_body(aq_h, ak_h, ae_h, m_h, src_h, dst_h,
                 aggp_o, sump_o,
                 srcv, dstv, aev, aqv, akv, pv, rowsv, zbuf, agg_sh, sum_sh,
                 sem, sem2, sem3):
    c = lax.axis_index("c")
    s = lax.axis_index("s")
    wid = s * _NC + c

    def _zrow(r, carry):
        for jj in range(8):
            rowsv[r, pl.ds(jj * 16, 16)] = jnp.zeros((16,), jnp.float32)
        return carry
    lax.fori_loop(0, _CH, _zrow, 0)

    def _zb(i, carry):
        zbuf[pl.ds(i * 16, 16)] = jnp.zeros((16,), jnp.float32)
        return carry
    lax.fori_loop(0, 64, _zb, 0)

    for k in range(5):
        nr = _CH if k < 4 else (_ROWS_PER_SUB - 4 * _CH)
        pltpu.sync_copy(rowsv.at[pl.ds(0, nr)],
                        agg_sh.at[pl.ds(s * _ROWS_PER_SUB + k * _CH, nr)])

    @pl.when(s == _NS - 1)
    def _():
        pltpu.sync_copy(rowsv.at[pl.ds(0, N_NODES - _NS * _ROWS_PER_SUB)],
                        agg_sh.at[pl.ds(_NS * _ROWS_PER_SUB,
                                        N_NODES - _NS * _ROWS_PER_SUB)])

    @pl.when(s < N_NODES // _SUM_PER_SUB)
    def _():
        pltpu.sync_copy(zbuf.at[pl.ds(0, 1000)],
                        sum_sh.at[pl.ds(s * _SUM_PER_SUB, _SUM_PER_SUB)])

    plsc.subcore_barrier()

    nj = _BASE_CHUNKS + jnp.where(wid < _EXTRA, 1, 0)

    def _chunk(j, carry):
        base = (wid + _NW * j) * _CH
        pltpu.sync_copy(src_h.at[pl.ds(base, _CH)], srcv)
        pltpu.sync_copy(dst_h.at[pl.ds(base, _CH)], dstv)
        pltpu.sync_copy(ae_h.at[pl.ds(base, _CH)], aev)
        cp1 = pltpu.async_copy(aq_h.at[dstv], aqv, sem)
        cp2 = pltpu.async_copy(ak_h.at[srcv], akv, sem2)
        cp3 = pltpu.async_copy(m_h.at[srcv], rowsv, sem3)
        cp1.wait()
        cp2.wait()
        cp3.wait()
        for jj in range(8):
            sl = pl.ds(jj * 16, 16)
            pv[sl] = jnp.exp(aqv[sl] + akv[sl] + aev[sl])

        def _scale(g, carry2):
            pvec = pv[pl.ds(g * 16, 16)]
            for u in range(16):
                pr = pvec[u]
                r = g * 16 + u
                for jj in range(8):
                    sl = pl.ds(jj * 16, 16)
                    rowsv[r, sl] = rowsv[r, sl] * pr
            return carry2
        lax.fori_loop(0, _CH // 16, _scale, 0)

        pltpu.sync_copy(rowsv, agg_sh.at[dstv], add=True)
        pltpu.sync_copy(pv, sum_sh.at[dstv], add=True)
        return carry
    lax.fori_loop(0, nj, _chunk, 0)

    plsc.subcore_barrier()

    for k in range(5):
        nr = _CH if k < 4 else (_ROWS_PER_SUB - 4 * _CH)
        r0 = s * _ROWS_PER_SUB + k * _CH
        pltpu.sync_copy(agg_sh.at[pl.ds(r0, nr)], rowsv.at[pl.ds(0, nr)])
        pltpu.sync_copy(rowsv.at[pl.ds(0, nr)], aggp_o.at[c, pl.ds(r0, nr)])

    _REM = N_NODES - _NS * _ROWS_PER_SUB

    @pl.when(s == _NS - 1)
    def _():
        pltpu.sync_copy(agg_sh.at[pl.ds(_NS * _ROWS_PER_SUB, _REM)],
                        rowsv.at[pl.ds(0, _REM)])
        pltpu.sync_copy(rowsv.at[pl.ds(0, _REM)],
                        aggp_o.at[c, pl.ds(_NS * _ROWS_PER_SUB, _REM)])

    @pl.when(s < N_NODES // _SUM_PER_SUB)
    def _():
        pltpu.sync_copy(sum_sh.at[pl.ds(s * _SUM_PER_SUB, _SUM_PER_SUB)],
                        zbuf.at[pl.ds(0, _SUM_PER_SUB)])
        pltpu.sync_copy(
            zbuf.at[pl.ds(0, _SUM_PER_SUB)],
            sump_o.at[pl.ds(c * N_NODES + s * _SUM_PER_SUB, _SUM_PER_SUB)])


def _sc_agg(aq, ak, ae, m, src, dst):
    mesh = plsc.VectorSubcoreMesh(core_axis_name="c", subcore_axis_name="s")
    f = functools.partial(
        pl.kernel,
        mesh=mesh,
        out_type=[
            jax.ShapeDtypeStruct((_NC, N_NODES, D), jnp.float32),
            jax.ShapeDtypeStruct((_NC * N_NODES,), jnp.float32),
        ],
        scratch_types=[
            pltpu.VMEM((_CH,), jnp.int32),
            pltpu.VMEM((_CH,), jnp.int32),
            pltpu.VMEM((_CH,), jnp.float32),
            pltpu.VMEM((_CH,), jnp.float32),
            pltpu.VMEM((_CH,), jnp.float32),
            pltpu.VMEM((_CH,), jnp.float32),
            pltpu.VMEM((_CH, D), jnp.float32),
            pltpu.VMEM((1024,), jnp.float32),
            pltpu.VMEM_SHARED((N_NODES, D), jnp.float32),
            pltpu.VMEM_SHARED((N_NODES,), jnp.float32),
            pltpu.SemaphoreType.DMA,
            pltpu.SemaphoreType.DMA,
            pltpu.SemaphoreType.DMA,
        ],
    )(_sc_agg_body)
    return f(aq, ak, ae, m, src, dst)


def _tc_final_body(aggp, sump, s_in, h, gamma, beta, w1, w2,
                   hn_o, u1_o, u2_o):
    denom = sump[0] + sump[1] + 1e-12
    agg = (aggp[0] + aggp[1]) / denom
    pre = jnp.maximum(s_in[...] + agg, 0.0) + h[...]
    mean = jnp.mean(pre, axis=-1, keepdims=True)
    cen = pre - mean
    var = jnp.mean(cen * cen, axis=-1, keepdims=True)
    hn = cen / jnp.sqrt(var + 1e-5) * gamma[...] + beta[...]
    hn_o[...] = hn
    u1_o[...] = _dot_t(hn, w1[...])
    u2_o[...] = _dot_t(hn, w2[...])


def _tc_final(aggp, sump, s_in, h, gamma, beta, w1, w2):
    blk = 2000
    grid = N_NODES // blk
    return pl.pallas_call(
        _tc_final_body,
        grid=(grid,),
        in_specs=[
            pl.BlockSpec((_NC, blk, D), lambda i: (0, i, 0)),
            pl.BlockSpec((_NC, blk, 1), lambda i: (0, i, 0)),
            pl.BlockSpec((blk, D), lambda i: (i, 0)),
            pl.BlockSpec((blk, D), lambda i: (i, 0)),
            pl.BlockSpec((1, D), lambda i: (0, 0)),
            pl.BlockSpec((1, D), lambda i: (0, 0)),
            pl.BlockSpec((D, D), lambda i: (0, 0)),
            pl.BlockSpec((D, D), lambda i: (0, 0)),
        ],
        out_specs=[
            pl.BlockSpec((blk, D), lambda i: (i, 0)),
            pl.BlockSpec((blk, D), lambda i: (i, 0)),
            pl.BlockSpec((blk, D), lambda i: (i, 0)),
        ],
        out_shape=[
            jax.ShapeDtypeStruct((N_NODES, D), jnp.float32),
            jax.ShapeDtypeStruct((N_NODES, D), jnp.float32),
            jax.ShapeDtypeStruct((N_NODES, D), jnp.float32),
        ],
    )(aggp, sump, s_in, h, gamma, beta, w1, w2)


def _sc_edge_body(u1_h, u2_h, e3_h, e_h, src_h, dst_h, enew_o,
                  srcv, dstv, u1v, u2v, e3v, ev, outv, sem, sem2):
    c = lax.axis_index("c")
    s = lax.axis_index("s")
    wid = s * _NC + c
    nj = _BASE_CHUNKS + jnp.where(wid < _EXTRA, 1, 0)

    def _chunk(j, carry):
        base = (wid + _NW * j) * _CH
        pltpu.sync_copy(src_h.at[pl.ds(base, _CH)], srcv)
        pltpu.sync_copy(dst_h.at[pl.ds(base, _CH)], dstv)
        pltpu.sync_copy(e3_h.at[pl.ds(base, _CH)], e3v)
        pltpu.sync_copy(e_h.at[pl.ds(base, _CH)], ev)
        cp1 = pltpu.async_copy(u1_h.at[srcv], u1v, sem)
        cp2 = pltpu.async_copy(u2_h.at[dstv], u2v, sem2)
        cp1.wait()
        cp2.wait()

        def _row(r, carry2):
            sl = pl.ds(0, ED)
            outv[r] = jnp.maximum(u1v[r, sl] + u2v[r, sl] + e3v[r], 0.0) + ev[r]
            return carry2
        lax.fori_loop(0, _CH, _row, 0)

        pltpu.sync_copy(outv, enew_o.at[pl.ds(base, _CH)])
        return carry
    lax.fori_loop(0, nj, _chunk, 0)


def _sc_edge(u1, u2, e3, e, src, dst):
    mesh = plsc.VectorSubcoreMesh(core_axis_name="c", subcore_axis_name="s")
    f = functools.partial(
        pl.kernel,
        mesh=mesh,
        out_type=jax.ShapeDtypeStruct((N_EDGES, ED), jnp.float32),
        scratch_types=[
            pltpu.VMEM((_CH,), jnp.int32),
            pltpu.VMEM((_CH,), jnp.int32),
            pltpu.VMEM((_CH, D), jnp.float32),
            pltpu.VMEM((_CH, D), jnp.float32),
            pltpu.VMEM((_CH, ED), jnp.float32),
            pltpu.VMEM((_CH, ED), jnp.float32),
            pltpu.VMEM((_CH, ED), jnp.float32),
            pltpu.SemaphoreType.DMA,
            pltpu.SemaphoreType.DMA,
        ],
    )(_sc_edge_body)
    return f(u1, u2, e3, e, src, dst)


def kernel(h, e, edge_index, W_q, W_k, W_e, attn_vec, W_self, b_self,
           W_msg, W_eu, b_eu, ln_gamma, ln_beta):
    src = edge_index[0]
    dst = edge_index[1]
    av = attn_vec[0]
    avq = av[:D].reshape(D, 1)
    avk = av[D:2 * D].reshape(D, 1)
    ave = av[2 * D:].reshape(D, 1)

    aq2, ak2, m, s_pre = _tc_node(h, W_q, W_k, W_msg, W_self,
                                  b_self.reshape(1, D), avq, avk)
    ae2, e3 = _tc_edge(e, W_e, ave, W_eu[:, 2 * D:], b_eu.reshape(1, ED))

    aggp, sump = _sc_agg(aq2.reshape(N_NODES), ak2.reshape(N_NODES),
                         ae2.reshape(N_EDGES), m, src, dst)

    w1p = jnp.zeros((D, D), jnp.float32).at[:ED].set(W_eu[:, :D])
    w2p = jnp.zeros((D, D), jnp.float32).at[:ED].set(W_eu[:, D:2 * D])
    h_new, u1, u2 = _tc_final(aggp, sump.reshape(_NC, N_NODES, 1), s_pre, h,
                              ln_gamma.reshape(1, D), ln_beta.reshape(1, D),
                              w1p, w2p)

    e_new = _sc_edge(u1, u2, e3, e, src, dst)
    return (h_new, e_new)


# speedup vs baseline: 6.0514x; 1.0810x over previous
import functools

import jax
import jax.numpy as jnp
from jax import lax
from jax.experimental import pallas as pl
from jax.experimental.pallas import tpu as pltpu, tpu_sc as plsc

N_NODES = 10000
N_EDGES = 320000
D = 128
ED = 16

_NC = 2
_NS = 16
_NW = _NC * _NS
_CH = 128
_NCHUNKS = N_EDGES // _CH
_BASE_CHUNKS = _NCHUNKS // _NW
_EXTRA = _NCHUNKS - _BASE_CHUNKS * _NW

_ROWS_PER_SUB = 624
_SUM_PER_SUB = 1000

_PREC = lax.Precision.DEFAULT


def _lrelu(x):
    return jnp.where(x >= 0, x, 0.2 * x)


def _dot_t(a, b):
    return lax.dot_general(a, b, (((1,), (1,)), ((), ())),
                           precision=_PREC, preferred_element_type=jnp.float32)


def _tc_node_body(h, wq, wk, wmsg, wself, bself, avq, avk,
                  aq_o, ak_o, m_o, s_o):
    hh = h[...]
    q = _dot_t(hh, wq[...])
    aq_o[...] = jnp.dot(_lrelu(q), avq[...], precision=_PREC,
                        preferred_element_type=jnp.float32)
    k = _dot_t(hh, wk[...])
    ak_o[...] = jnp.dot(_lrelu(k), avk[...], precision=_PREC,
                        preferred_element_type=jnp.float32)
    m_o[...] = _dot_t(hh, wmsg[...])
    s_o[...] = _dot_t(hh, wself[...]) + bself[...]


def _tc_node(h, wq, wk, wmsg, wself, bself, avq, avk):
    blk = 2000
    grid = N_NODES // blk
    return pl.pallas_call(
        _tc_node_body,
        grid=(grid,),
        in_specs=[
            pl.BlockSpec((blk, D), lambda i: (i, 0)),
            pl.BlockSpec((D, D), lambda i: (0, 0)),
            pl.BlockSpec((D, D), lambda i: (0, 0)),
            pl.BlockSpec((D, D), lambda i: (0, 0)),
            pl.BlockSpec((D, D), lambda i: (0, 0)),
            pl.BlockSpec((1, D), lambda i: (0, 0)),
            pl.BlockSpec((D, 1), lambda i: (0, 0)),
            pl.BlockSpec((D, 1), lambda i: (0, 0)),
        ],
        out_specs=[
            pl.BlockSpec((blk, 1), lambda i: (i, 0)),
            pl.BlockSpec((blk, 1), lambda i: (i, 0)),
            pl.BlockSpec((blk, D), lambda i: (i, 0)),
            pl.BlockSpec((blk, D), lambda i: (i, 0)),
        ],
        out_shape=[
            jax.ShapeDtypeStruct((N_NODES, 1), jnp.float32),
            jax.ShapeDtypeStruct((N_NODES, 1), jnp.float32),
            jax.ShapeDtypeStruct((N_NODES, D), jnp.float32),
            jax.ShapeDtypeStruct((N_NODES, D), jnp.float32),
        ],
    )(h, wq, wk, wmsg, wself, bself, avq, avk)


def _tc_edge_body(e, we, ave, w3, beu, ae_o, e3_o):
    eb = e[...]
    p = _dot_t(eb, we[...])
    ae_o[...] = jnp.dot(_lrelu(p), ave[...], precision=_PREC,
                        preferred_element_type=jnp.float32)
    e3_o[...] = _dot_t(eb, w3[...]) + beu[...]


def _tc_edge(e, we, ave, w3, beu):
    blk = 8000
    grid = N_EDGES // blk
    return pl.pallas_call(
        _tc_edge_body,
        grid=(grid,),
        in_specs=[
            pl.BlockSpec((blk, ED), lambda i: (i, 0)),
            pl.BlockSpec((D, ED), lambda i: (0, 0)),
            pl.BlockSpec((D, 1), lambda i: (0, 0)),
            pl.BlockSpec((ED, ED), lambda i: (0, 0)),
            pl.BlockSpec((1, ED), lambda i: (0, 0)),
        ],
        out_specs=[
            pl.BlockSpec((blk, 1), lambda i: (i, 0)),
            pl.BlockSpec((blk, ED), lambda i: (i, 0)),
        ],
        out_shape=[
            jax.ShapeDtypeStruct((N_EDGES, 1), jnp.float32),
            jax.ShapeDtypeStruct((N_EDGES, ED), jnp.float32),
        ],
    )(e, we, ave, w3, beu)


def _sc_agg_body(aq_h, ak_h, ae_h, m_h, src_h, dst_h,
                 aggp_o, sump_o,
                 srcv, dstv, aev, aqv, akv, pv, rowsv,
                 srcv2, dstv2, aev2, aqv2, akv2, pv2, rowsv2,
                 zbuf, agg_sh, sum_sh,
                 sem, sem2, sem3, sem4, sem5, sem6):
    c = lax.axis_index("c")
    s = lax.axis_index("s")
    wid = s * _NC + c

    def _zrow(r, carry):
        for jj in range(8):
            rowsv[r, pl.ds(jj * 16, 16)] = jnp.zeros((16,), jnp.float32)
        return carry
    lax.fori_loop(0, _CH, _zrow, 0)

    def _zb(i, carry):
        zbuf[pl.ds(i * 16, 16)] = jnp.zeros((16,), jnp.float32)
        return carry
    lax.fori_loop(0, 64, _zb, 0)

    for k in range(5):
        nr = _CH if k < 4 else (_ROWS_PER_SUB - 4 * _CH)
        pltpu.sync_copy(rowsv.at[pl.ds(0, nr)],
                        agg_sh.at[pl.ds(s * _ROWS_PER_SUB + k * _CH, nr)])

    @pl.when(s == _NS - 1)
    def _():
        pltpu.sync_copy(rowsv.at[pl.ds(0, N_NODES - _NS * _ROWS_PER_SUB)],
                        agg_sh.at[pl.ds(_NS * _ROWS_PER_SUB,
                                        N_NODES - _NS * _ROWS_PER_SUB)])

    @pl.when(s < N_NODES // _SUM_PER_SUB)
    def _():
        pltpu.sync_copy(zbuf.at[pl.ds(0, 1000)],
                        sum_sh.at[pl.ds(s * _SUM_PER_SUB, _SUM_PER_SUB)])

    plsc.subcore_barrier()

    nj = _BASE_CHUNKS + jnp.where(wid < _EXTRA, 1, 0)

    bufs = ((srcv, dstv, aev, aqv, akv, pv, rowsv, sem, sem2, sem3),
            (srcv2, dstv2, aev2, aqv2, akv2, pv2, rowsv2, sem4, sem5, sem6))

    def _fire(j, b):
        sv, dv, av_, qv, kv, pv_, rv, s1, s2, s3 = b
        base = (wid + _NW * j) * _CH
        pltpu.sync_copy(src_h.at[pl.ds(base, _CH)], sv)
        pltpu.sync_copy(dst_h.at[pl.ds(base, _CH)], dv)
        pltpu.sync_copy(ae_h.at[pl.ds(base, _CH)], av_)
        pltpu.async_copy(aq_h.at[dv], qv, s1)
        pltpu.async_copy(ak_h.at[sv], kv, s2)
        pltpu.async_copy(m_h.at[sv], rv, s3)

    def _proc(b):
        sv, dv, av_, qv, kv, pv_, rv, s1, s2, s3 = b
        pltpu.make_async_copy(aq_h.at[dv], qv, s1).wait()
        pltpu.make_async_copy(ak_h.at[sv], kv, s2).wait()
        pltpu.make_async_copy(m_h.at[sv], rv, s3).wait()
        for jj in range(8):
            sl = pl.ds(jj * 16, 16)
            pv_[sl] = jnp.exp(qv[sl] + kv[sl] + av_[sl])

        def _scale(g, carry2):
            pvec = pv_[pl.ds(g * 16, 16)]
            for u in range(16):
                pr = pvec[u]
                r = g * 16 + u
                for jj in range(8):
                    sl = pl.ds(jj * 16, 16)
                    rv[r, sl] = rv[r, sl] * pr
            return carry2
        lax.fori_loop(0, _CH // 16, _scale, 0)

        pltpu.sync_copy(rv, agg_sh.at[dv], add=True)
        pltpu.sync_copy(pv_, sum_sh.at[dv], add=True)

    _fire(0, bufs[0])

    def _chunk(t, carry):
        even = (t % 2) == 0
        more = t + 1 < nj

        @pl.when(jnp.logical_and(more, even))
        def _():
            _fire(t + 1, bufs[1])

        @pl.when(jnp.logical_and(more, jnp.logical_not(even)))
        def _():
            _fire(t + 1, bufs[0])

        @pl.when(even)
        def _():
            _proc(bufs[0])

        @pl.when(jnp.logical_not(even))
        def _():
            _proc(bufs[1])
        return carry
    lax.fori_loop(0, nj, _chunk, 0)

    plsc.subcore_barrier()

    for k in range(5):
        nr = _CH if k < 4 else (_ROWS_PER_SUB - 4 * _CH)
        r0 = s * _ROWS_PER_SUB + k * _CH
        pltpu.sync_copy(agg_sh.at[pl.ds(r0, nr)], rowsv.at[pl.ds(0, nr)])
        pltpu.sync_copy(rowsv.at[pl.ds(0, nr)], aggp_o.at[c, pl.ds(r0, nr)])

    _REM = N_NODES - _NS * _ROWS_PER_SUB

    @pl.when(s == _NS - 1)
    def _():
        pltpu.sync_copy(agg_sh.at[pl.ds(_NS * _ROWS_PER_SUB, _REM)],
                        rowsv.at[pl.ds(0, _REM)])
        pltpu.sync_copy(rowsv.at[pl.ds(0, _REM)],
                        aggp_o.at[c, pl.ds(_NS * _ROWS_PER_SUB, _REM)])

    @pl.when(s < N_NODES // _SUM_PER_SUB)
    def _():
        pltpu.sync_copy(sum_sh.at[pl.ds(s * _SUM_PER_SUB, _SUM_PER_SUB)],
                        zbuf.at[pl.ds(0, _SUM_PER_SUB)])
        pltpu.sync_copy(
            zbuf.at[pl.ds(0, _SUM_PER_SUB)],
            sump_o.at[pl.ds(c * N_NODES + s * _SUM_PER_SUB, _SUM_PER_SUB)])


def _sc_agg(aq, ak, ae, m, src, dst):
    mesh = plsc.VectorSubcoreMesh(core_axis_name="c", subcore_axis_name="s")
    f = functools.partial(
        pl.kernel,
        mesh=mesh,
        out_type=[
            jax.ShapeDtypeStruct((_NC, N_NODES, D), jnp.float32),
            jax.ShapeDtypeStruct((_NC * N_NODES,), jnp.float32),
        ],
        scratch_types=(
            [pltpu.VMEM((_CH,), jnp.int32),
             pltpu.VMEM((_CH,), jnp.int32),
             pltpu.VMEM((_CH,), jnp.float32),
             pltpu.VMEM((_CH,), jnp.float32),
             pltpu.VMEM((_CH,), jnp.float32),
             pltpu.VMEM((_CH,), jnp.float32),
             pltpu.VMEM((_CH, D), jnp.float32)] * 2
            + [pltpu.VMEM((1024,), jnp.float32),
               pltpu.VMEM_SHARED((N_NODES, D), jnp.float32),
               pltpu.VMEM_SHARED((N_NODES,), jnp.float32)]
            + [pltpu.SemaphoreType.DMA] * 6
        ),
    )(_sc_agg_body)
    return f(aq, ak, ae, m, src, dst)


def _tc_final_body(aggp, sump, s_in, h, gamma, beta, w1, w2,
                   hn_o, u1_o, u2_o):
    denom = sump[0] + sump[1] + 1e-12
    agg = (aggp[0] + aggp[1]) / denom
    pre = jnp.maximum(s_in[...] + agg, 0.0) + h[...]
    mean = jnp.mean(pre, axis=-1, keepdims=True)
    cen = pre - mean
    var = jnp.mean(cen * cen, axis=-1, keepdims=True)
    hn = cen / jnp.sqrt(var + 1e-5) * gamma[...] + beta[...]
    hn_o[...] = hn
    u1_o[...] = _dot_t(hn, w1[...])
    u2_o[...] = _dot_t(hn, w2[...])


def _tc_final(aggp, sump, s_in, h, gamma, beta, w1, w2):
    blk = 2000
    grid = N_NODES // blk
    return pl.pallas_call(
        _tc_final_body,
        grid=(grid,),
        in_specs=[
            pl.BlockSpec((_NC, blk, D), lambda i: (0, i, 0)),
            pl.BlockSpec((_NC, blk, 1), lambda i: (0, i, 0)),
            pl.BlockSpec((blk, D), lambda i: (i, 0)),
            pl.BlockSpec((blk, D), lambda i: (i, 0)),
            pl.BlockSpec((1, D), lambda i: (0, 0)),
            pl.BlockSpec((1, D), lambda i: (0, 0)),
            pl.BlockSpec((D, D), lambda i: (0, 0)),
            pl.BlockSpec((D, D), lambda i: (0, 0)),
        ],
        out_specs=[
            pl.BlockSpec((blk, D), lambda i: (i, 0)),
            pl.BlockSpec((blk, D), lambda i: (i, 0)),
            pl.BlockSpec((blk, D), lambda i: (i, 0)),
        ],
        out_shape=[
            jax.ShapeDtypeStruct((N_NODES, D), jnp.float32),
            jax.ShapeDtypeStruct((N_NODES, D), jnp.float32),
            jax.ShapeDtypeStruct((N_NODES, D), jnp.float32),
        ],
    )(aggp, sump, s_in, h, gamma, beta, w1, w2)


def _sc_edge_body(u1_h, u2_h, e3_h, e_h, src_h, dst_h, enew_o,
                  srcv, dstv, u1v, u2v, e3v, ev, outv, sem, sem2):
    c = lax.axis_index("c")
    s = lax.axis_index("s")
    wid = s * _NC + c
    nj = _BASE_CHUNKS + jnp.where(wid < _EXTRA, 1, 0)

    def _chunk(j, carry):
        base = (wid + _NW * j) * _CH
        pltpu.sync_copy(src_h.at[pl.ds(base, _CH)], srcv)
        pltpu.sync_copy(dst_h.at[pl.ds(base, _CH)], dstv)
        pltpu.sync_copy(e3_h.at[pl.ds(base, _CH)], e3v)
        pltpu.sync_copy(e_h.at[pl.ds(base, _CH)], ev)
        cp1 = pltpu.async_copy(u1_h.at[srcv], u1v, sem)
        cp2 = pltpu.async_copy(u2_h.at[dstv], u2v, sem2)
        cp1.wait()
        cp2.wait()

        def _row(r, carry2):
            sl = pl.ds(0, ED)
            outv[r] = jnp.maximum(u1v[r, sl] + u2v[r, sl] + e3v[r], 0.0) + ev[r]
            return carry2
        lax.fori_loop(0, _CH, _row, 0)

        pltpu.sync_copy(outv, enew_o.at[pl.ds(base, _CH)])
        return carry
    lax.fori_loop(0, nj, _chunk, 0)


def _sc_edge(u1, u2, e3, e, src, dst):
    mesh = plsc.VectorSubcoreMesh(core_axis_name="c", subcore_axis_name="s")
    f = functools.partial(
        pl.kernel,
        mesh=mesh,
        out_type=jax.ShapeDtypeStruct((N_EDGES, ED), jnp.float32),
        scratch_types=[
            pltpu.VMEM((_CH,), jnp.int32),
            pltpu.VMEM((_CH,), jnp.int32),
            pltpu.VMEM((_CH, D), jnp.float32),
            pltpu.VMEM((_CH, D), jnp.float32),
            pltpu.VMEM((_CH, ED), jnp.float32),
            pltpu.VMEM((_CH, ED), jnp.float32),
            pltpu.VMEM((_CH, ED), jnp.float32),
            pltpu.SemaphoreType.DMA,
            pltpu.SemaphoreType.DMA,
        ],
    )(_sc_edge_body)
    return f(u1, u2, e3, e, src, dst)


def kernel(h, e, edge_index, W_q, W_k, W_e, attn_vec, W_self, b_self,
           W_msg, W_eu, b_eu, ln_gamma, ln_beta):
    src = edge_index[0]
    dst = edge_index[1]
    av = attn_vec[0]
    avq = av[:D].reshape(D, 1)
    avk = av[D:2 * D].reshape(D, 1)
    ave = av[2 * D:].reshape(D, 1)

    aq2, ak2, m, s_pre = _tc_node(h, W_q, W_k, W_msg, W_self,
                                  b_self.reshape(1, D), avq, avk)
    ae2, e3 = _tc_edge(e, W_e, ave, W_eu[:, 2 * D:], b_eu.reshape(1, ED))

    aggp, sump = _sc_agg(aq2.reshape(N_NODES), ak2.reshape(N_NODES),
                         ae2.reshape(N_EDGES), m, src, dst)

    w1p = jnp.zeros((D, D), jnp.float32).at[:ED].set(W_eu[:, :D])
    w2p = jnp.zeros((D, D), jnp.float32).at[:ED].set(W_eu[:, D:2 * D])
    h_new, u1, u2 = _tc_final(aggp, sump.reshape(_NC, N_NODES, 1), s_pre, h,
                              ln_gamma.reshape(1, D), ln_beta.reshape(1, D),
                              w1p, w2p)

    e_new = _sc_edge(u1, u2, e3, e, src, dst)
    return (h_new, e_new)


# speedup vs baseline: 7.7394x; 1.2789x over previous
import functools

import jax
import jax.numpy as jnp
from jax import lax
from jax.experimental import pallas as pl
from jax.experimental.pallas import tpu as pltpu, tpu_sc as plsc

N_NODES = 10000
N_EDGES = 320000
D = 128
ED = 16

_NC = 2
_NS = 16
_NW = _NC * _NS
_CH = 128
_NCHUNKS = N_EDGES // _CH
_BASE_CHUNKS = _NCHUNKS // _NW
_EXTRA = _NCHUNKS - _BASE_CHUNKS * _NW

_ROWS_PER_SUB = 624
_SUM_PER_SUB = 1000

_PREC = lax.Precision.DEFAULT


def _lrelu(x):
    return jnp.where(x >= 0, x, 0.2 * x)


def _dot_t(a, b):
    return lax.dot_general(a, b, (((1,), (1,)), ((), ())),
                           precision=_PREC, preferred_element_type=jnp.float32)


def _tc_node_body(h, wq, wk, wmsg, wself, bself, avq, avk,
                  aq_o, ak_o, m_o, s_o):
    hh = h[...]
    q = _dot_t(hh, wq[...])
    aq_o[...] = jnp.dot(_lrelu(q), avq[...], precision=_PREC,
                        preferred_element_type=jnp.float32)
    k = _dot_t(hh, wk[...])
    ak_o[...] = jnp.dot(_lrelu(k), avk[...], precision=_PREC,
                        preferred_element_type=jnp.float32)
    m_o[...] = _dot_t(hh, wmsg[...])
    s_o[...] = _dot_t(hh, wself[...]) + bself[...]


def _tc_node(h, wq, wk, wmsg, wself, bself, avq, avk):
    blk = 2000
    grid = N_NODES // blk
    return pl.pallas_call(
        _tc_node_body,
        grid=(grid,),
        in_specs=[
            pl.BlockSpec((blk, D), lambda i: (i, 0)),
            pl.BlockSpec((D, D), lambda i: (0, 0)),
            pl.BlockSpec((D, D), lambda i: (0, 0)),
            pl.BlockSpec((D, D), lambda i: (0, 0)),
            pl.BlockSpec((D, D), lambda i: (0, 0)),
            pl.BlockSpec((1, D), lambda i: (0, 0)),
            pl.BlockSpec((D, 1), lambda i: (0, 0)),
            pl.BlockSpec((D, 1), lambda i: (0, 0)),
        ],
        out_specs=[
            pl.BlockSpec((blk, 1), lambda i: (i, 0)),
            pl.BlockSpec((blk, 1), lambda i: (i, 0)),
            pl.BlockSpec((blk, D), lambda i: (i, 0)),
            pl.BlockSpec((blk, D), lambda i: (i, 0)),
        ],
        out_shape=[
            jax.ShapeDtypeStruct((N_NODES, 1), jnp.float32),
            jax.ShapeDtypeStruct((N_NODES, 1), jnp.float32),
            jax.ShapeDtypeStruct((N_NODES, D), jnp.float32),
            jax.ShapeDtypeStruct((N_NODES, D), jnp.float32),
        ],
    )(h, wq, wk, wmsg, wself, bself, avq, avk)


def _tc_edge_body(e, we, ave, w3, beu, ae_o, e3_o):
    eb = e[...]
    p = _dot_t(eb, we[...])
    ae_o[...] = jnp.dot(_lrelu(p), ave[...], precision=_PREC,
                        preferred_element_type=jnp.float32)
    e3_o[...] = jnp.concatenate([_dot_t(eb, w3[...]) + beu[...], eb], axis=1)


def _tc_edge(e, we, ave, w3, beu):
    blk = 8000
    grid = N_EDGES // blk
    return pl.pallas_call(
        _tc_edge_body,
        grid=(grid,),
        in_specs=[
            pl.BlockSpec((blk, ED), lambda i: (i, 0)),
            pl.BlockSpec((D, ED), lambda i: (0, 0)),
            pl.BlockSpec((D, 1), lambda i: (0, 0)),
            pl.BlockSpec((ED, ED), lambda i: (0, 0)),
            pl.BlockSpec((1, ED), lambda i: (0, 0)),
        ],
        out_specs=[
            pl.BlockSpec((blk, 1), lambda i: (i, 0)),
            pl.BlockSpec((blk, 2 * ED), lambda i: (i, 0)),
        ],
        out_shape=[
            jax.ShapeDtypeStruct((N_EDGES, 1), jnp.float32),
            jax.ShapeDtypeStruct((N_EDGES, 2 * ED), jnp.float32),
        ],
    )(e, we, ave, w3, beu)


def _sc_agg_body(aq_h, ak_h, ae_h, m_h, src_h, dst_h,
                 aggp_o, sump_o,
                 srcv, dstv, aev, aqv, akv, pv, rowsv,
                 srcv2, dstv2, aev2, aqv2, akv2, pv2, rowsv2,
                 zbuf, agg_sh, sum_sh,
                 sem, sem2, sem3, sem4, sem5, sem6):
    c = lax.axis_index("c")
    s = lax.axis_index("s")
    wid = s * _NC + c

    def _zrow(r, carry):
        for jj in range(8):
            rowsv[r, pl.ds(jj * 16, 16)] = jnp.zeros((16,), jnp.float32)
        return carry
    lax.fori_loop(0, _CH, _zrow, 0)

    def _zb(i, carry):
        zbuf[pl.ds(i * 16, 16)] = jnp.zeros((16,), jnp.float32)
        return carry
    lax.fori_loop(0, 64, _zb, 0)

    for k in range(5):
        nr = _CH if k < 4 else (_ROWS_PER_SUB - 4 * _CH)
        pltpu.sync_copy(rowsv.at[pl.ds(0, nr)],
                        agg_sh.at[pl.ds(s * _ROWS_PER_SUB + k * _CH, nr)])

    @pl.when(s == _NS - 1)
    def _():
        pltpu.sync_copy(rowsv.at[pl.ds(0, N_NODES - _NS * _ROWS_PER_SUB)],
                        agg_sh.at[pl.ds(_NS * _ROWS_PER_SUB,
                                        N_NODES - _NS * _ROWS_PER_SUB)])

    @pl.when(s < N_NODES // _SUM_PER_SUB)
    def _():
        pltpu.sync_copy(zbuf.at[pl.ds(0, 1000)],
                        sum_sh.at[pl.ds(s * _SUM_PER_SUB, _SUM_PER_SUB)])

    plsc.subcore_barrier()

    nj = _BASE_CHUNKS + jnp.where(wid < _EXTRA, 1, 0)

    bufs = ((srcv, dstv, aev, aqv, akv, pv, rowsv, sem, sem2, sem3),
            (srcv2, dstv2, aev2, aqv2, akv2, pv2, rowsv2, sem4, sem5, sem6))

    def _fire(j, b):
        sv, dv, av_, qv, kv, pv_, rv, s1, s2, s3 = b
        base = (wid + _NW * j) * _CH
        pltpu.sync_copy(src_h.at[pl.ds(base, _CH)], sv)
        pltpu.sync_copy(dst_h.at[pl.ds(base, _CH)], dv)
        pltpu.sync_copy(ae_h.at[pl.ds(base, _CH)], av_)
        pltpu.async_copy(aq_h.at[dv], qv, s1)
        pltpu.async_copy(ak_h.at[sv], kv, s2)
        pltpu.async_copy(m_h.at[sv], rv, s3)

    def _proc(b):
        sv, dv, av_, qv, kv, pv_, rv, s1, s2, s3 = b
        pltpu.make_async_copy(aq_h.at[dv], qv, s1).wait()
        pltpu.make_async_copy(ak_h.at[sv], kv, s2).wait()
        pltpu.make_async_copy(m_h.at[sv], rv, s3).wait()
        for jj in range(8):
            sl = pl.ds(jj * 16, 16)
            pv_[sl] = jnp.exp(qv[sl] + kv[sl] + av_[sl])

        def _scale(g, carry2):
            pvec = pv_[pl.ds(g * 16, 16)]
            for u in range(16):
                pr = pvec[u]
                r = g * 16 + u
                for jj in range(8):
                    sl = pl.ds(jj * 16, 16)
                    rv[r, sl] = rv[r, sl] * pr
            return carry2
        lax.fori_loop(0, _CH // 16, _scale, 0)

        pltpu.sync_copy(rv, agg_sh.at[dv], add=True)
        pltpu.sync_copy(pv_, sum_sh.at[dv], add=True)

    _fire(0, bufs[0])

    def _chunk(t, carry):
        even = (t % 2) == 0
        more = t + 1 < nj

        @pl.when(jnp.logical_and(more, even))
        def _():
            _fire(t + 1, bufs[1])

        @pl.when(jnp.logical_and(more, jnp.logical_not(even)))
        def _():
            _fire(t + 1, bufs[0])

        @pl.when(even)
        def _():
            _proc(bufs[0])

        @pl.when(jnp.logical_not(even))
        def _():
            _proc(bufs[1])
        return carry
    lax.fori_loop(0, nj, _chunk, 0)

    plsc.subcore_barrier()

    for k in range(5):
        nr = _CH if k < 4 else (_ROWS_PER_SUB - 4 * _CH)
        r0 = s * _ROWS_PER_SUB + k * _CH
        pltpu.sync_copy(agg_sh.at[pl.ds(r0, nr)], rowsv.at[pl.ds(0, nr)])
        pltpu.sync_copy(rowsv.at[pl.ds(0, nr)], aggp_o.at[c, pl.ds(r0, nr)])

    _REM = N_NODES - _NS * _ROWS_PER_SUB

    @pl.when(s == _NS - 1)
    def _():
        pltpu.sync_copy(agg_sh.at[pl.ds(_NS * _ROWS_PER_SUB, _REM)],
                        rowsv.at[pl.ds(0, _REM)])
        pltpu.sync_copy(rowsv.at[pl.ds(0, _REM)],
                        aggp_o.at[c, pl.ds(_NS * _ROWS_PER_SUB, _REM)])

    @pl.when(s < N_NODES // _SUM_PER_SUB)
    def _():
        pltpu.sync_copy(sum_sh.at[pl.ds(s * _SUM_PER_SUB, _SUM_PER_SUB)],
                        zbuf.at[pl.ds(0, _SUM_PER_SUB)])
        pltpu.sync_copy(
            zbuf.at[pl.ds(0, _SUM_PER_SUB)],
            sump_o.at[pl.ds(c * N_NODES + s * _SUM_PER_SUB, _SUM_PER_SUB)])


def _sc_agg(aq, ak, ae, m, src, dst):
    mesh = plsc.VectorSubcoreMesh(core_axis_name="c", subcore_axis_name="s")
    f = functools.partial(
        pl.kernel,
        mesh=mesh,
        out_type=[
            jax.ShapeDtypeStruct((_NC, N_NODES, D), jnp.float32),
            jax.ShapeDtypeStruct((_NC * N_NODES,), jnp.float32),
        ],
        scratch_types=(
            [pltpu.VMEM((_CH,), jnp.int32),
             pltpu.VMEM((_CH,), jnp.int32),
             pltpu.VMEM((_CH,), jnp.float32),
             pltpu.VMEM((_CH,), jnp.float32),
             pltpu.VMEM((_CH,), jnp.float32),
             pltpu.VMEM((_CH,), jnp.float32),
             pltpu.VMEM((_CH, D), jnp.float32)] * 2
            + [pltpu.VMEM((1024,), jnp.float32),
               pltpu.VMEM_SHARED((N_NODES, D), jnp.float32),
               pltpu.VMEM_SHARED((N_NODES,), jnp.float32)]
            + [pltpu.SemaphoreType.DMA] * 6
        ),
    )(_sc_agg_body)
    return f(aq, ak, ae, m, src, dst)


def _tc_final_body(aggp, sump, s_in, h, gamma, beta, w1, w2,
                   hn_o, u1_o, u2_o):
    denom = sump[0] + sump[1] + 1e-12
    agg = (aggp[0] + aggp[1]) / denom
    pre = jnp.maximum(s_in[...] + agg, 0.0) + h[...]
    mean = jnp.mean(pre, axis=-1, keepdims=True)
    cen = pre - mean
    var = jnp.mean(cen * cen, axis=-1, keepdims=True)
    hn = cen / jnp.sqrt(var + 1e-5) * gamma[...] + beta[...]
    hn_o[...] = hn
    u1_o[...] = _dot_t(hn, w1[...])
    u2_o[...] = _dot_t(hn, w2[...])


def _tc_final(aggp, sump, s_in, h, gamma, beta, w1, w2):
    blk = 2000
    grid = N_NODES // blk
    return pl.pallas_call(
        _tc_final_body,
        grid=(grid,),
        in_specs=[
            pl.BlockSpec((_NC, blk, D), lambda i: (0, i, 0)),
            pl.BlockSpec((_NC, blk, 1), lambda i: (0, i, 0)),
            pl.BlockSpec((blk, D), lambda i: (i, 0)),
            pl.BlockSpec((blk, D), lambda i: (i, 0)),
            pl.BlockSpec((1, D), lambda i: (0, 0)),
            pl.BlockSpec((1, D), lambda i: (0, 0)),
            pl.BlockSpec((D, D), lambda i: (0, 0)),
            pl.BlockSpec((D, D), lambda i: (0, 0)),
        ],
        out_specs=[
            pl.BlockSpec((blk, D), lambda i: (i, 0)),
            pl.BlockSpec((blk, D), lambda i: (i, 0)),
            pl.BlockSpec((blk, D), lambda i: (i, 0)),
        ],
        out_shape=[
            jax.ShapeDtypeStruct((N_NODES, D), jnp.float32),
            jax.ShapeDtypeStruct((N_NODES, D), jnp.float32),
            jax.ShapeDtypeStruct((N_NODES, D), jnp.float32),
        ],
    )(aggp, sump, s_in, h, gamma, beta, w1, w2)


def _sc_edge_body(u1_h, u2_h, x_h, src_h, dst_h, out_o,
                  srcv, dstv, u1v, u2v, xv,
                  srcv2, dstv2, u1v2, u2v2, xv2,
                  sem, sem2, sem3, sem4):
    c = lax.axis_index("c")
    s = lax.axis_index("s")
    wid = s * _NC + c
    nj = _BASE_CHUNKS + jnp.where(wid < _EXTRA, 1, 0)

    bufs = ((srcv, dstv, u1v, u2v, xv, sem, sem2),
            (srcv2, dstv2, u1v2, u2v2, xv2, sem3, sem4))

    def _fire(j, b):
        sv, dv, u1, u2, xx, s1, s2 = b
        base = (wid + _NW * j) * _CH
        pltpu.sync_copy(src_h.at[pl.ds(base, _CH)], sv)
        pltpu.sync_copy(dst_h.at[pl.ds(base, _CH)], dv)
        pltpu.sync_copy(x_h.at[pl.ds(base, _CH)], xx)
        pltpu.async_copy(u1_h.at[sv], u1, s1)
        pltpu.async_copy(u2_h.at[dv], u2, s2)

    def _proc(j, b):
        sv, dv, u1, u2, xx, s1, s2 = b
        base = (wid + _NW * j) * _CH
        pltpu.make_async_copy(u1_h.at[sv], u1, s1).wait()
        pltpu.make_async_copy(u2_h.at[dv], u2, s2).wait()

        def _row(r, carry2):
            sl = pl.ds(0, ED)
            sl2 = pl.ds(ED, ED)
            xx[r, sl] = (jnp.maximum(u1[r, sl] + u2[r, sl] + xx[r, sl], 0.0)
                         + xx[r, sl2])
            return carry2
        lax.fori_loop(0, _CH, _row, 0)

        pltpu.sync_copy(xx, out_o.at[pl.ds(base, _CH)])

    _fire(0, bufs[0])

    def _chunk(t, carry):
        even = (t % 2) == 0
        more = t + 1 < nj

        @pl.when(jnp.logical_and(more, even))
        def _():
            _fire(t + 1, bufs[1])

        @pl.when(jnp.logical_and(more, jnp.logical_not(even)))
        def _():
            _fire(t + 1, bufs[0])

        @pl.when(even)
        def _():
            _proc(t, bufs[0])

        @pl.when(jnp.logical_not(even))
        def _():
            _proc(t, bufs[1])
        return carry
    lax.fori_loop(0, nj, _chunk, 0)


def _sc_edge(u1, u2, x, src, dst):
    mesh = plsc.VectorSubcoreMesh(core_axis_name="c", subcore_axis_name="s")
    f = functools.partial(
        pl.kernel,
        mesh=mesh,
        out_type=jax.ShapeDtypeStruct((N_EDGES, 2 * ED), jnp.float32),
        scratch_types=(
            [pltpu.VMEM((_CH,), jnp.int32),
             pltpu.VMEM((_CH,), jnp.int32),
             pltpu.VMEM((_CH, D), jnp.float32),
             pltpu.VMEM((_CH, D), jnp.float32),
             pltpu.VMEM((_CH, 2 * ED), jnp.float32)] * 2
            + [pltpu.SemaphoreType.DMA] * 4
        ),
    )(_sc_edge_body)
    return f(u1, u2, x, src, dst)


def kernel(h, e, edge_index, W_q, W_k, W_e, attn_vec, W_self, b_self,
           W_msg, W_eu, b_eu, ln_gamma, ln_beta):
    src = edge_index[0]
    dst = edge_index[1]
    av = attn_vec[0]
    avq = av[:D].reshape(D, 1)
    avk = av[D:2 * D].reshape(D, 1)
    ave = av[2 * D:].reshape(D, 1)

    aq2, ak2, m, s_pre = _tc_node(h, W_q, W_k, W_msg, W_self,
                                  b_self.reshape(1, D), avq, avk)
    ae2, e3 = _tc_edge(e, W_e, ave, W_eu[:, 2 * D:], b_eu.reshape(1, ED))

    aggp, sump = _sc_agg(aq2.reshape(N_NODES), ak2.reshape(N_NODES),
                         ae2.reshape(N_EDGES), m, src, dst)

    w1p = jnp.zeros((D, D), jnp.float32).at[:ED].set(W_eu[:, :D])
    w2p = jnp.zeros((D, D), jnp.float32).at[:ED].set(W_eu[:, D:2 * D])
    h_new, u1, u2 = _tc_final(aggp, sump.reshape(_NC, N_NODES, 1), s_pre, h,
                              ln_gamma.reshape(1, D), ln_beta.reshape(1, D),
                              w1p, w2p)

    out32 = _sc_edge(u1, u2, e3, src, dst)
    return (h_new, out32[:, :ED])
